# baseline (speedup 1.0000x reference)
import jax, jax.numpy as jnp
import numpy as np
import math
from jax.experimental import pallas as pl

INPUT_DIM = 192
DIM = 96
NUM_ANGLE = 90
NUM_RHO = 90
N, H, W = 2, 224, 224


def _rho_table(numAngle, numRho, Hh, Ww):
    irho = int(math.sqrt(Hh * Hh + Ww * Ww) + 1) / float(numRho - 1)
    itheta = math.pi / numAngle
    ang = np.arange(numAngle) * itheta
    tabCos = np.cos(ang) / irho
    tabSin = np.sin(ang) / irho
    xs = (np.arange(Ww) - Ww // 2).astype(np.float64)
    ys = (np.arange(Hh) - Hh // 2).astype(np.float64)
    r = np.round(xs[None, None, :] * tabCos[:, None, None] + ys[None, :, None] * tabSin[:, None, None])
    r = r.astype(np.int64) + numRho // 2
    r = np.clip(r, 0, numRho - 1)
    return r


_RIDX = jnp.asarray(_rho_table(NUM_ANGLE, NUM_RHO, H, W), dtype=jnp.int32)
_SIDX = (jnp.arange(NUM_ANGLE, dtype=jnp.int32)[:, None] * NUM_RHO
         + _RIDX.reshape(NUM_ANGLE, H * W)).reshape(-1)


def _dht(x):
    n, c, h, w = x.shape
    flat = x.reshape(n * c, h * w)

    def one(v):
        vals = jnp.broadcast_to(v[None, :], (NUM_ANGLE, h * w)).reshape(-1)
        return jnp.zeros(NUM_ANGLE * NUM_RHO, x.dtype).at[_SIDX].add(vals)

    acc = jax.vmap(one)(flat)
    return acc.reshape(n, c, NUM_ANGLE, NUM_RHO)


def _conv2d(x, w, b, pad):
    out = jax.lax.conv_general_dilated(x, w, window_strides=(1, 1), padding=pad,
                                       dimension_numbers=('NCHW', 'OIHW', 'NCHW'))
    return out + b[None, :, None, None]


def _bn(x, g, b, eps=1e-5):
    mean = jnp.mean(x, axis=(0, 2, 3), keepdims=True)
    var = jnp.var(x, axis=(0, 2, 3), keepdims=True)
    xh = (x - mean) / jnp.sqrt(var + eps)
    return xh * g[None, :, None, None] + b[None, :, None, None]


def _identity_pallas(x):
    def body(x_ref, o_ref):
        o_ref[...] = x_ref[...]
    return pl.pallas_call(body, out_shape=jax.ShapeDtypeStruct(x.shape, x.dtype))(x)


def kernel(x, W1, b1, g1, be1, W2, b2, g2, be2, W3, b3, g3, be3):
    h1 = jax.nn.relu(_bn(_conv2d(x, W1, b1, 'VALID'), g1, be1))
    acc = _dht(h1)
    h2 = jax.nn.relu(_bn(_conv2d(acc, W2, b2, 'SAME'), g2, be2))
    h3 = jax.nn.relu(_bn(_conv2d(h2, W3, b3, 'SAME'), g3, be3))
    return _identity_pallas(h3)


# speedup vs baseline: 9.3106x; 9.3106x over previous
import functools
import math

import numpy as np
import jax
import jax.numpy as jnp
from jax import lax
from jax.experimental import pallas as pl
from jax.experimental.pallas import tpu as pltpu
from jax.experimental.pallas import tpu_sc as plsc

INPUT_DIM = 192
DIM = 96
NUM_ANGLE = 90
NUM_RHO = 90
N, H, W = 2, 224, 224
PW = 256
NSL = N * DIM
RB = 6
EPS = 1e-5
HIGH = lax.Precision.HIGHEST

PG = 92
NPIX = PG * PG
PADEXT = 96
EXT = NPIX + 2 * PADEXT
CH = 4232
NCH = NPIX // CH


def _build_tables():
    irho = int(math.sqrt(H * H + W * W) + 1) / float(NUM_RHO - 1)
    itheta = math.pi / NUM_ANGLE
    ang = np.arange(NUM_ANGLE) * itheta
    tabCos = np.cos(ang) / irho
    tabSin = np.sin(ang) / irho
    xs = (np.arange(W) - W // 2).astype(np.float64)
    ys = (np.arange(H) - H // 2).astype(np.float64)
    r = np.round(xs[None, None, :] * tabCos[:, None, None]
                 + ys[None, :, None] * tabSin[:, None, None])
    ridx = np.clip(r.astype(np.int64) + NUM_RHO // 2, 0, NUM_RHO - 1)

    rng = np.arange(NUM_RHO + 1)
    bnd_ext = np.zeros((NUM_ANGLE, 6 * 16, H), dtype=np.int64)
    sgn = np.zeros(NUM_ANGLE, dtype=np.float32)
    for a in range(NUM_ANGLE):
        cnt = (ridx[a][:, :, None] < rng[None, None, :]).sum(1)
        if tabCos[a] >= 0:
            sgn[a] = 1.0
            bnd = cnt
        else:
            sgn[a] = -1.0
            bnd = W - cnt
        bnd_ext[a, :NUM_RHO + 1, :] = bnd.T
    idx = bnd_ext.reshape(NUM_ANGLE, RB, 16, H).transpose(0, 1, 3, 2)
    idx = idx + (np.arange(H) * PW)[None, None, :, None]
    idx = idx.reshape(NUM_ANGLE, RB * H * 16).astype(np.int32)
    return idx, sgn


_IDX_NP, _SGN_NP = _build_tables()
_IDX = jnp.asarray(_IDX_NP)
_SGN = jnp.asarray(_SGN_NP.reshape(NUM_ANGLE, 1))

_U_NP = ((np.arange(W)[:, None] + 1) <= np.arange(PW)[None, :]) & \
        (np.arange(PW)[None, :] <= W)
_U = jnp.asarray(_U_NP.astype(np.float32))

_qi = np.arange(NPIX) // PG
_qj = np.arange(NPIX) % PG
_valid_q = ((_qi >= 1) & (_qi <= 90) & (_qj >= 1) & (_qj <= 90))
_MVALID = jnp.asarray(_valid_q.astype(np.float32).reshape(NPIX, 1))
_MVALID_EXT = jnp.asarray(
    np.pad(_valid_q.astype(np.float32), (PADEXT, PADEXT)).reshape(EXT, 1))


def _k1_body(x_ref, c_ref, s_ref):
    m = pl.program_id(0)
    xb = x_ref[0]
    c = lax.dot_general(xb, xb, (((1,), (1,)), ((), ())),
                        preferred_element_type=jnp.float32, precision=HIGH)
    s = jnp.sum(xb, axis=1, keepdims=True)

    @pl.when(m == 0)
    def _():
        c_ref[...] = c
        s_ref[...] = s

    @pl.when(m > 0)
    def _():
        c_ref[...] += c
        s_ref[...] += s


def _k1(xr):
    return pl.pallas_call(
        _k1_body,
        grid=(N * 7,),
        in_specs=[pl.BlockSpec((1, INPUT_DIM, 7168),
                               lambda m: (m // 7, 0, m % 7))],
        out_specs=[pl.BlockSpec((INPUT_DIM, INPUT_DIM), lambda m: (0, 0)),
                   pl.BlockSpec((INPUT_DIM, 1), lambda m: (0, 0))],
        out_shape=[jax.ShapeDtypeStruct((INPUT_DIM, INPUT_DIM), jnp.float32),
                   jax.ShapeDtypeStruct((INPUT_DIM, 1), jnp.float32)],
    )(xr)


def _k2a_body(x_ref, w_ref, sc_ref, sh_ref, o_ref):
    xb = x_ref[0]
    z = lax.dot_general(w_ref[...], xb, (((1,), (0,)), ((), ())),
                        preferred_element_type=jnp.float32)
    o_ref[0] = jnp.maximum(z * sc_ref[...] + sh_ref[...], 0.0)


def _k2a(xr, w1f, sc1, sh1):
    return pl.pallas_call(
        _k2a_body,
        grid=(N, 7),
        in_specs=[pl.BlockSpec((1, INPUT_DIM, 7168), lambda n, h: (n, 0, h)),
                  pl.BlockSpec((DIM, INPUT_DIM), lambda n, h: (0, 0)),
                  pl.BlockSpec((DIM, 1), lambda n, h: (0, 0)),
                  pl.BlockSpec((DIM, 1), lambda n, h: (0, 0))],
        out_specs=pl.BlockSpec((1, DIM, 7168), lambda n, h: (n, 0, h)),
        out_shape=jax.ShapeDtypeStruct((N, DIM, H * W), jnp.float32),
    )(xr, w1f, sc1, sh1)


def _k2b_body(a_ref, u_ref, o_ref):
    u = u_ref[...]
    for i in range(8):
        o_ref[i] = lax.dot_general(a_ref[i], u, (((1,), (0,)), ((), ())),
                                   preferred_element_type=jnp.float32,
                                   precision=HIGH)


def _k2b(h1r):
    return pl.pallas_call(
        _k2b_body,
        grid=(NSL // 8,),
        in_specs=[pl.BlockSpec((8, H, W), lambda m: (m, 0, 0)),
                  pl.BlockSpec((W, PW), lambda m: (0, 0))],
        out_specs=pl.BlockSpec((8, H, PW), lambda m: (m, 0, 0)),
        out_shape=jax.ShapeDtypeStruct((NSL, H, PW), jnp.float32),
    )(h1r, _U)


_IDX_ROW = RB * H * 16


def _dht_sc_body(p_hbm, idx_hbm, t_hbm, p_v, i0_v, i1_v, t_v,
                 sem_p, sem_i0, sem_i1, sem_t):
    nc = 2
    wid = lax.axis_index("s") * nc + lax.axis_index("c")

    def process_angle(a, buf):
        def rbody(rb, _):
            def ybody(y, acc):
                idx16 = buf[pl.ds((rb * H + y) * 16, 16)]
                return acc + plsc.load_gather(p_v, [idx16])
            acc = lax.fori_loop(0, H, ybody, jnp.zeros((16,), jnp.float32),
                                unroll=8)
            t_v[pl.ds((a * RB + rb) * 16, 16)] = acc
            return 0
        lax.fori_loop(0, RB, rbody, 0)

    for si in range(NSL // 32):
        s = si * 32 + wid
        cp = pltpu.make_async_copy(p_hbm.at[s], p_v, sem_p)
        cp.start()
        cp.wait()
        pltpu.make_async_copy(idx_hbm.at[0], i0_v, sem_i0).start()

        def pair(k, _):
            a0 = 2 * k
            pltpu.make_async_copy(idx_hbm.at[a0], i0_v, sem_i0).wait()
            pltpu.make_async_copy(idx_hbm.at[a0 + 1], i1_v, sem_i1).start()
            process_angle(a0, i0_v)
            pltpu.make_async_copy(idx_hbm.at[a0 + 1], i1_v, sem_i1).wait()

            @pl.when(a0 + 2 < NUM_ANGLE)
            def _():
                pltpu.make_async_copy(idx_hbm.at[a0 + 2], i0_v,
                                      sem_i0).start()
            process_angle(a0 + 1, i1_v)
            return 0

        lax.fori_loop(0, NUM_ANGLE // 2, pair, 0)
        ct = pltpu.make_async_copy(t_v, t_hbm.at[s], sem_t)
        ct.start()
        ct.wait()


def _dht_sc(pf):
    mesh = plsc.VectorSubcoreMesh(core_axis_name="c", subcore_axis_name="s")
    f = functools.partial(
        pl.kernel,
        mesh=mesh,
        compiler_params=pltpu.CompilerParams(needs_layout_passes=False),
        out_type=jax.ShapeDtypeStruct((NSL, NUM_ANGLE * RB * 16),
                                      jnp.float32),
        scratch_types=[
            pltpu.VMEM((H * PW,), jnp.float32),
            pltpu.VMEM((_IDX_ROW,), jnp.int32),
            pltpu.VMEM((_IDX_ROW,), jnp.int32),
            pltpu.VMEM((NUM_ANGLE * RB * 16,), jnp.float32),
            pltpu.SemaphoreType.DMA,
            pltpu.SemaphoreType.DMA,
            pltpu.SemaphoreType.DMA,
            pltpu.SemaphoreType.DMA,
        ],
    )(_dht_sc_body)
    return f(pf, _IDX)


def _kdiff_body(t0_ref, t1_ref, sg_ref, o_ref):
    o_ref[...] = (t1_ref[...] - t0_ref[...]) * sg_ref[...][None, :, :]


def _kdiff(t0, t1):
    return pl.pallas_call(
        _kdiff_body,
        grid=(4,),
        in_specs=[pl.BlockSpec((NSL // 4, NUM_ANGLE, NUM_RHO),
                               lambda m: (m, 0, 0)),
                  pl.BlockSpec((NSL // 4, NUM_ANGLE, NUM_RHO),
                               lambda m: (m, 0, 0)),
                  pl.BlockSpec((NUM_ANGLE, 1), lambda m: (0, 0))],
        out_specs=pl.BlockSpec((NSL // 4, NUM_ANGLE, NUM_RHO),
                               lambda m: (m, 0, 0)),
        out_shape=jax.ShapeDtypeStruct((NSL, NUM_ANGLE, NUM_RHO),
                                       jnp.float32),
    )(t0, t1, _SGN)


def _conv_chunk(n, c, load_fn, w_ref, b_ref, mo_ref, o_ref, s_ref, q_ref):
    cbase = c * CH
    o = None
    for ki in range(3):
        for kj in range(3):
            base = cbase + PADEXT + (ki - 1) * PG + (kj - 1)
            sl = load_fn(base)
            d = lax.dot_general(sl, w_ref[ki * 3 + kj],
                                (((1,), (0,)), ((), ())),
                                preferred_element_type=jnp.float32)
            o = d if o is None else o + d
    o = o + b_ref[...]
    mo = mo_ref[...]
    so = jnp.sum(o * mo, axis=0, keepdims=True)
    qo = jnp.sum(o * o * mo, axis=0, keepdims=True)
    o_ref[0] = o
    first = jnp.logical_and(n == 0, c == 0)

    @pl.when(first)
    def _():
        s_ref[...] = so
        q_ref[...] = qo

    @pl.when(jnp.logical_not(first))
    def _():
        s_ref[...] += so
        q_ref[...] += qo


def _conv2_body(a_ref, w_ref, b_ref, mo_ref, o_ref, s_ref, q_ref):
    n, c = pl.program_id(0), pl.program_id(1)

    def load(base):
        return a_ref[0, pl.ds(base, CH), :]
    _conv_chunk(n, c, load, w_ref, b_ref, mo_ref, o_ref, s_ref, q_ref)


def _conv3_body(a_ref, w_ref, b_ref, sc_ref, sh_ref, mi_ref, mo_ref,
                o_ref, s_ref, q_ref):
    n, c = pl.program_id(0), pl.program_id(1)
    sc, sh = sc_ref[...], sh_ref[...]

    def load(base):
        sl = a_ref[0, pl.ds(base, CH), :]
        mi = mi_ref[pl.ds(base, CH), :]
        return jnp.maximum(sl * sc + sh, 0.0) * mi
    _conv_chunk(n, c, load, w_ref, b_ref, mo_ref, o_ref, s_ref, q_ref)


_CONV_OUT = [jax.ShapeDtypeStruct((N, NPIX, DIM), jnp.float32),
             jax.ShapeDtypeStruct((1, DIM), jnp.float32),
             jax.ShapeDtypeStruct((1, DIM), jnp.float32)]
_CONV_OUT_SPECS = [pl.BlockSpec((1, CH, DIM), lambda n, c: (n, c, 0)),
                   pl.BlockSpec((1, DIM), lambda n, c: (0, 0)),
                   pl.BlockSpec((1, DIM), lambda n, c: (0, 0))]


def _kconv2(a2p, w2r, b2c):
    return pl.pallas_call(
        _conv2_body,
        grid=(N, NCH),
        in_specs=[pl.BlockSpec((1, EXT, DIM), lambda n, c: (n, 0, 0)),
                  pl.BlockSpec((9, DIM, DIM), lambda n, c: (0, 0, 0)),
                  pl.BlockSpec((1, DIM), lambda n, c: (0, 0)),
                  pl.BlockSpec((CH, 1), lambda n, c: (c, 0))],
        out_specs=_CONV_OUT_SPECS,
        out_shape=_CONV_OUT,
    )(a2p, w2r, b2c, _MVALID)


def _kconv3(o2p, w3r, b3c, sc2c, sh2c):
    return pl.pallas_call(
        _conv3_body,
        grid=(N, NCH),
        in_specs=[pl.BlockSpec((1, EXT, DIM), lambda n, c: (n, 0, 0)),
                  pl.BlockSpec((9, DIM, DIM), lambda n, c: (0, 0, 0)),
                  pl.BlockSpec((1, DIM), lambda n, c: (0, 0)),
                  pl.BlockSpec((1, DIM), lambda n, c: (0, 0)),
                  pl.BlockSpec((1, DIM), lambda n, c: (0, 0)),
                  pl.BlockSpec((EXT, 1), lambda n, c: (0, 0)),
                  pl.BlockSpec((CH, 1), lambda n, c: (c, 0))],
        out_specs=_CONV_OUT_SPECS,
        out_shape=_CONV_OUT,
    )(o2p, w3r, b3c, sc2c, sh2c, _MVALID_EXT, _MVALID)


def _kfinal_body(a_ref, sc_ref, sh_ref, o_ref):
    o_ref[...] = jnp.maximum(a_ref[...] * sc_ref[...] + sh_ref[...], 0.0)


def _kfinal(h3, sc3c, sh3c):
    return pl.pallas_call(
        _kfinal_body,
        grid=(N,),
        in_specs=[pl.BlockSpec((1, NPIX, DIM), lambda n: (n, 0, 0)),
                  pl.BlockSpec((1, DIM), lambda n: (0, 0)),
                  pl.BlockSpec((1, DIM), lambda n: (0, 0))],
        out_specs=pl.BlockSpec((1, NPIX, DIM), lambda n: (n, 0, 0)),
        out_shape=jax.ShapeDtypeStruct((N, NPIX, DIM), jnp.float32),
    )(h3, sc3c, sh3c)


def _pad_rows(o):
    z = jnp.zeros((N, PADEXT, DIM), jnp.float32)
    return jnp.concatenate([z, o, z], axis=1)


def _bn_fold(s, q, g, be, npx):
    mean = s[0] / npx
    var = q[0] / npx - mean * mean
    sc = g * lax.rsqrt(var + EPS)
    sh = be - mean * sc
    return sc[None, :], sh[None, :]


def kernel(x, W1, b1, g1, be1, W2, b2, g2, be2, W3, b3, g3, be3):
    xr = x.reshape(N, INPUT_DIM, H * W)

    C, S = _k1(xr)
    npx1 = float(N * H * W)
    mu = S[:, 0] / npx1
    Cc = C / npx1 - mu[:, None] * mu[None, :]
    W1f = W1.reshape(DIM, INPUT_DIM)
    m1 = W1f @ mu + b1
    var1 = jnp.sum((W1f @ Cc) * W1f, axis=1)
    sc1 = g1 * lax.rsqrt(var1 + EPS)
    sh1 = (b1 - m1) * sc1 + be1

    h1 = _k2a(xr, W1f, sc1[:, None], sh1[:, None])
    h1r = h1.reshape(NSL, H, W)
    P = _k2b(h1r)
    T = _dht_sc(P.reshape(NSL, H * PW))

    T3 = T.reshape(NSL, NUM_ANGLE, RB * 16)
    acc = _kdiff(T3[:, :, 0:NUM_RHO], T3[:, :, 1:NUM_RHO + 1])

    a2 = acc.reshape(N, DIM, NUM_ANGLE * NUM_RHO).transpose(0, 2, 1)
    a2 = a2.reshape(N, NUM_ANGLE, NUM_RHO, DIM)
    a2p = jnp.pad(a2, ((0, 0), (1, 1), (1, 1), (0, 0))).reshape(N, NPIX, DIM)

    w2r = W2.transpose(2, 3, 1, 0).reshape(9, DIM, DIM)
    o2, s2, q2 = _kconv2(_pad_rows(a2p), w2r, b2[None, :])
    sc2, sh2 = _bn_fold(s2, q2, g2, be2, float(N * NUM_ANGLE * NUM_RHO))

    w3r = W3.transpose(2, 3, 1, 0).reshape(9, DIM, DIM)
    o3, s3, q3 = _kconv3(_pad_rows(o2), w3r, b3[None, :], sc2, sh2)
    sc3, sh3 = _bn_fold(s3, q3, g3, be3, float(N * NUM_ANGLE * NUM_RHO))

    h3 = _kfinal(o3, sc3, sh3)
    h3f = h3.reshape(N, PG, PG, DIM)[:, 1:91, 1:91, :]
    return h3f.transpose(0, 3, 1, 2)


# speedup vs baseline: 9.4681x; 1.0169x over previous
import functools
import math

import numpy as np
import jax
import jax.numpy as jnp
from jax import lax
from jax.experimental import pallas as pl
from jax.experimental.pallas import tpu as pltpu
from jax.experimental.pallas import tpu_sc as plsc

INPUT_DIM = 192
DIM = 96
NUM_ANGLE = 90
NUM_RHO = 90
N, H, W = 2, 224, 224
PW = 256
NSL = N * DIM
RB = 6
EPS = 1e-5
HIGH = lax.Precision.HIGHEST

PG = 92
NPIX = PG * PG
PADEXT = 96
EXT = NPIX + 2 * PADEXT
CH = 4232
NCH = NPIX // CH


def _build_tables():
    irho = int(math.sqrt(H * H + W * W) + 1) / float(NUM_RHO - 1)
    itheta = math.pi / NUM_ANGLE
    ang = np.arange(NUM_ANGLE) * itheta
    tabCos = np.cos(ang) / irho
    tabSin = np.sin(ang) / irho
    xs = (np.arange(W) - W // 2).astype(np.float64)
    ys = (np.arange(H) - H // 2).astype(np.float64)
    r = np.round(xs[None, None, :] * tabCos[:, None, None]
                 + ys[None, :, None] * tabSin[:, None, None])
    ridx = np.clip(r.astype(np.int64) + NUM_RHO // 2, 0, NUM_RHO - 1)

    rng = np.arange(NUM_RHO + 1)
    bnd_ext = np.zeros((NUM_ANGLE, 6 * 16, H), dtype=np.int64)
    sgn = np.zeros(NUM_ANGLE, dtype=np.float32)
    for a in range(NUM_ANGLE):
        cnt = (ridx[a][:, :, None] < rng[None, None, :]).sum(1)
        if tabCos[a] >= 0:
            sgn[a] = 1.0
            bnd = cnt
        else:
            sgn[a] = -1.0
            bnd = W - cnt
        bnd_ext[a, :NUM_RHO + 1, :] = bnd.T
    idx = bnd_ext.reshape(NUM_ANGLE, RB, 16, H).transpose(0, 1, 3, 2)
    idx = idx + (np.arange(H) * PW)[None, None, :, None]
    idx = idx.reshape(NUM_ANGLE, RB, H // 2, 2, 16).astype(np.int64)
    packed = idx[:, :, :, 0, :] | (idx[:, :, :, 1, :] << 16)
    packed = packed.astype(np.uint32).view(np.int32)
    return packed.reshape(NUM_ANGLE, RB * (H // 2) * 16), sgn


_IDX_NP, _SGN_NP = _build_tables()
_IDX = jnp.asarray(_IDX_NP)
_SGN = jnp.asarray(_SGN_NP.reshape(NUM_ANGLE, 1))

_U_NP = ((np.arange(W)[:, None] + 1) <= np.arange(PW)[None, :]) & \
        (np.arange(PW)[None, :] <= W)
_U = jnp.asarray(_U_NP.astype(np.float32))

_qi = np.arange(NPIX) // PG
_qj = np.arange(NPIX) % PG
_valid_q = ((_qi >= 1) & (_qi <= 90) & (_qj >= 1) & (_qj <= 90))
_MVALID = jnp.asarray(_valid_q.astype(np.float32).reshape(NPIX, 1))
_MVALID_EXT = jnp.asarray(
    np.pad(_valid_q.astype(np.float32), (PADEXT, PADEXT)).reshape(EXT, 1))


def _k1_body(x_ref, c_ref, s_ref):
    m = pl.program_id(0)
    xb = x_ref[0]
    c = lax.dot_general(xb, xb, (((1,), (1,)), ((), ())),
                        preferred_element_type=jnp.float32, precision=HIGH)
    s = jnp.sum(xb, axis=1, keepdims=True)

    @pl.when(m == 0)
    def _():
        c_ref[...] = c
        s_ref[...] = s

    @pl.when(m > 0)
    def _():
        c_ref[...] += c
        s_ref[...] += s


def _k1(xr):
    return pl.pallas_call(
        _k1_body,
        grid=(N * 7,),
        in_specs=[pl.BlockSpec((1, INPUT_DIM, 7168),
                               lambda m: (m // 7, 0, m % 7))],
        out_specs=[pl.BlockSpec((INPUT_DIM, INPUT_DIM), lambda m: (0, 0)),
                   pl.BlockSpec((INPUT_DIM, 1), lambda m: (0, 0))],
        out_shape=[jax.ShapeDtypeStruct((INPUT_DIM, INPUT_DIM), jnp.float32),
                   jax.ShapeDtypeStruct((INPUT_DIM, 1), jnp.float32)],
    )(xr)


def _k2a_body(x_ref, w_ref, sc_ref, sh_ref, o_ref):
    xb = x_ref[0]
    z = lax.dot_general(w_ref[...], xb, (((1,), (0,)), ((), ())),
                        preferred_element_type=jnp.float32)
    o_ref[0] = jnp.maximum(z * sc_ref[...] + sh_ref[...], 0.0)


def _k2a(xr, w1f, sc1, sh1):
    return pl.pallas_call(
        _k2a_body,
        grid=(N, 7),
        in_specs=[pl.BlockSpec((1, INPUT_DIM, 7168), lambda n, h: (n, 0, h)),
                  pl.BlockSpec((DIM, INPUT_DIM), lambda n, h: (0, 0)),
                  pl.BlockSpec((DIM, 1), lambda n, h: (0, 0)),
                  pl.BlockSpec((DIM, 1), lambda n, h: (0, 0))],
        out_specs=pl.BlockSpec((1, DIM, 7168), lambda n, h: (n, 0, h)),
        out_shape=jax.ShapeDtypeStruct((N, DIM, H * W), jnp.float32),
    )(xr, w1f, sc1, sh1)


def _k2b_body(a_ref, u_ref, o_ref):
    u = u_ref[...]
    for i in range(8):
        o_ref[i] = lax.dot_general(a_ref[i], u, (((1,), (0,)), ((), ())),
                                   preferred_element_type=jnp.float32,
                                   precision=HIGH)


def _k2b(h1r):
    return pl.pallas_call(
        _k2b_body,
        grid=(NSL // 8,),
        in_specs=[pl.BlockSpec((8, H, W), lambda m: (m, 0, 0)),
                  pl.BlockSpec((W, PW), lambda m: (0, 0))],
        out_specs=pl.BlockSpec((8, H, PW), lambda m: (m, 0, 0)),
        out_shape=jax.ShapeDtypeStruct((NSL, H, PW), jnp.float32),
    )(h1r, _U)


_IDX_ROW = RB * (H // 2) * 16


def _dht_sc_body(p_hbm, idx_hbm, t_hbm, p_v, i0_v, i1_v, t_v,
                 sem_p, sem_i0, sem_i1, sem_t):
    nc = 2
    wid = lax.axis_index("s") * nc + lax.axis_index("c")
    m16 = jnp.full((16,), 0xFFFF, jnp.int32)

    def process_angle(a, buf):
        def rbody(rb, _):
            def ybody(y2, acc):
                v = buf[pl.ds((rb * (H // 2) + y2) * 16, 16)]
                acc = acc + plsc.load_gather(p_v, [v & m16])
                return acc + plsc.load_gather(
                    p_v, [lax.shift_right_logical(v, 16)])
            acc = lax.fori_loop(0, H // 2, ybody,
                                jnp.zeros((16,), jnp.float32), unroll=8)
            t_v[pl.ds((a * RB + rb) * 16, 16)] = acc
            return 0
        lax.fori_loop(0, RB, rbody, 0)

    for si in range(NSL // 32):
        s = si * 32 + wid
        cp = pltpu.make_async_copy(p_hbm.at[s], p_v, sem_p)
        cp.start()
        cp.wait()
        pltpu.make_async_copy(idx_hbm.at[0], i0_v, sem_i0).start()

        def pair(k, _):
            a0 = 2 * k
            pltpu.make_async_copy(idx_hbm.at[a0], i0_v, sem_i0).wait()
            pltpu.make_async_copy(idx_hbm.at[a0 + 1], i1_v, sem_i1).start()
            process_angle(a0, i0_v)
            pltpu.make_async_copy(idx_hbm.at[a0 + 1], i1_v, sem_i1).wait()

            @pl.when(a0 + 2 < NUM_ANGLE)
            def _():
                pltpu.make_async_copy(idx_hbm.at[a0 + 2], i0_v,
                                      sem_i0).start()
            process_angle(a0 + 1, i1_v)
            return 0

        lax.fori_loop(0, NUM_ANGLE // 2, pair, 0)
        ct = pltpu.make_async_copy(t_v, t_hbm.at[s], sem_t)
        ct.start()
        ct.wait()


def _dht_sc(pf):
    mesh = plsc.VectorSubcoreMesh(core_axis_name="c", subcore_axis_name="s")
    f = functools.partial(
        pl.kernel,
        mesh=mesh,
        compiler_params=pltpu.CompilerParams(needs_layout_passes=False),
        out_type=jax.ShapeDtypeStruct((NSL, NUM_ANGLE * RB * 16),
                                      jnp.float32),
        scratch_types=[
            pltpu.VMEM((H * PW,), jnp.float32),
            pltpu.VMEM((_IDX_ROW,), jnp.int32),
            pltpu.VMEM((_IDX_ROW,), jnp.int32),
            pltpu.VMEM((NUM_ANGLE * RB * 16,), jnp.float32),
            pltpu.SemaphoreType.DMA,
            pltpu.SemaphoreType.DMA,
            pltpu.SemaphoreType.DMA,
            pltpu.SemaphoreType.DMA,
        ],
    )(_dht_sc_body)
    return f(pf, _IDX)


def _kdiff_body(t0_ref, t1_ref, sg_ref, o_ref):
    o_ref[...] = (t1_ref[...] - t0_ref[...]) * sg_ref[...][None, :, :]


def _kdiff(t0, t1):
    return pl.pallas_call(
        _kdiff_body,
        grid=(4,),
        in_specs=[pl.BlockSpec((NSL // 4, NUM_ANGLE, NUM_RHO),
                               lambda m: (m, 0, 0)),
                  pl.BlockSpec((NSL // 4, NUM_ANGLE, NUM_RHO),
                               lambda m: (m, 0, 0)),
                  pl.BlockSpec((NUM_ANGLE, 1), lambda m: (0, 0))],
        out_specs=pl.BlockSpec((NSL // 4, NUM_ANGLE, NUM_RHO),
                               lambda m: (m, 0, 0)),
        out_shape=jax.ShapeDtypeStruct((NSL, NUM_ANGLE, NUM_RHO),
                                       jnp.float32),
    )(t0, t1, _SGN)


def _conv_chunk(n, c, load_fn, w_ref, b_ref, mo_ref, o_ref, s_ref, q_ref):
    cbase = c * CH
    o = None
    for ki in range(3):
        for kj in range(3):
            base = cbase + PADEXT + (ki - 1) * PG + (kj - 1)
            sl = load_fn(base)
            d = lax.dot_general(sl, w_ref[ki * 3 + kj],
                                (((1,), (0,)), ((), ())),
                                preferred_element_type=jnp.float32)
            o = d if o is None else o + d
    o = o + b_ref[...]
    mo = mo_ref[...]
    so = jnp.sum(o * mo, axis=0, keepdims=True)
    qo = jnp.sum(o * o * mo, axis=0, keepdims=True)
    o_ref[0] = o
    first = jnp.logical_and(n == 0, c == 0)

    @pl.when(first)
    def _():
        s_ref[...] = so
        q_ref[...] = qo

    @pl.when(jnp.logical_not(first))
    def _():
        s_ref[...] += so
        q_ref[...] += qo


def _conv2_body(a_ref, w_ref, b_ref, mo_ref, o_ref, s_ref, q_ref):
    n, c = pl.program_id(0), pl.program_id(1)

    def load(base):
        return a_ref[0, pl.ds(base, CH), :]
    _conv_chunk(n, c, load, w_ref, b_ref, mo_ref, o_ref, s_ref, q_ref)


def _conv3_body(a_ref, w_ref, b_ref, sc_ref, sh_ref, mi_ref, mo_ref,
                o_ref, s_ref, q_ref):
    n, c = pl.program_id(0), pl.program_id(1)
    sc, sh = sc_ref[...], sh_ref[...]

    def load(base):
        sl = a_ref[0, pl.ds(base, CH), :]
        mi = mi_ref[pl.ds(base, CH), :]
        return jnp.maximum(sl * sc + sh, 0.0) * mi
    _conv_chunk(n, c, load, w_ref, b_ref, mo_ref, o_ref, s_ref, q_ref)


_CONV_OUT = [jax.ShapeDtypeStruct((N, NPIX, DIM), jnp.float32),
             jax.ShapeDtypeStruct((1, DIM), jnp.float32),
             jax.ShapeDtypeStruct((1, DIM), jnp.float32)]
_CONV_OUT_SPECS = [pl.BlockSpec((1, CH, DIM), lambda n, c: (n, c, 0)),
                   pl.BlockSpec((1, DIM), lambda n, c: (0, 0)),
                   pl.BlockSpec((1, DIM), lambda n, c: (0, 0))]


def _kconv2(a2p, w2r, b2c):
    return pl.pallas_call(
        _conv2_body,
        grid=(N, NCH),
        in_specs=[pl.BlockSpec((1, EXT, DIM), lambda n, c: (n, 0, 0)),
                  pl.BlockSpec((9, DIM, DIM), lambda n, c: (0, 0, 0)),
                  pl.BlockSpec((1, DIM), lambda n, c: (0, 0)),
                  pl.BlockSpec((CH, 1), lambda n, c: (c, 0))],
        out_specs=_CONV_OUT_SPECS,
        out_shape=_CONV_OUT,
    )(a2p, w2r, b2c, _MVALID)


def _kconv3(o2p, w3r, b3c, sc2c, sh2c):
    return pl.pallas_call(
        _conv3_body,
        grid=(N, NCH),
        in_specs=[pl.BlockSpec((1, EXT, DIM), lambda n, c: (n, 0, 0)),
                  pl.BlockSpec((9, DIM, DIM), lambda n, c: (0, 0, 0)),
                  pl.BlockSpec((1, DIM), lambda n, c: (0, 0)),
                  pl.BlockSpec((1, DIM), lambda n, c: (0, 0)),
                  pl.BlockSpec((1, DIM), lambda n, c: (0, 0)),
                  pl.BlockSpec((EXT, 1), lambda n, c: (0, 0)),
                  pl.BlockSpec((CH, 1), lambda n, c: (c, 0))],
        out_specs=_CONV_OUT_SPECS,
        out_shape=_CONV_OUT,
    )(o2p, w3r, b3c, sc2c, sh2c, _MVALID_EXT, _MVALID)


def _kfinal_body(a_ref, sc_ref, sh_ref, o_ref):
    o_ref[...] = jnp.maximum(a_ref[...] * sc_ref[...] + sh_ref[...], 0.0)


def _kfinal(h3, sc3c, sh3c):
    return pl.pallas_call(
        _kfinal_body,
        grid=(N,),
        in_specs=[pl.BlockSpec((1, NPIX, DIM), lambda n: (n, 0, 0)),
                  pl.BlockSpec((1, DIM), lambda n: (0, 0)),
                  pl.BlockSpec((1, DIM), lambda n: (0, 0))],
        out_specs=pl.BlockSpec((1, NPIX, DIM), lambda n: (n, 0, 0)),
        out_shape=jax.ShapeDtypeStruct((N, NPIX, DIM), jnp.float32),
    )(h3, sc3c, sh3c)


def _pad_rows(o):
    z = jnp.zeros((N, PADEXT, DIM), jnp.float32)
    return jnp.concatenate([z, o, z], axis=1)


def _bn_fold(s, q, g, be, npx):
    mean = s[0] / npx
    var = q[0] / npx - mean * mean
    sc = g * lax.rsqrt(var + EPS)
    sh = be - mean * sc
    return sc[None, :], sh[None, :]


def kernel(x, W1, b1, g1, be1, W2, b2, g2, be2, W3, b3, g3, be3):
    xr = x.reshape(N, INPUT_DIM, H * W)

    C, S = _k1(xr)
    npx1 = float(N * H * W)
    mu = S[:, 0] / npx1
    Cc = C / npx1 - mu[:, None] * mu[None, :]
    W1f = W1.reshape(DIM, INPUT_DIM)
    m1 = W1f @ mu + b1
    var1 = jnp.sum((W1f @ Cc) * W1f, axis=1)
    sc1 = g1 * lax.rsqrt(var1 + EPS)
    sh1 = (b1 - m1) * sc1 + be1

    h1 = _k2a(xr, W1f, sc1[:, None], sh1[:, None])
    h1r = h1.reshape(NSL, H, W)
    P = _k2b(h1r)
    T = _dht_sc(P.reshape(NSL, H * PW))

    T3 = T.reshape(NSL, NUM_ANGLE, RB * 16)
    acc = _kdiff(T3[:, :, 0:NUM_RHO], T3[:, :, 1:NUM_RHO + 1])

    a2 = acc.reshape(N, DIM, NUM_ANGLE * NUM_RHO).transpose(0, 2, 1)
    a2 = a2.reshape(N, NUM_ANGLE, NUM_RHO, DIM)
    a2p = jnp.pad(a2, ((0, 0), (1, 1), (1, 1), (0, 0))).reshape(N, NPIX, DIM)

    w2r = W2.transpose(2, 3, 1, 0).reshape(9, DIM, DIM)
    o2, s2, q2 = _kconv2(_pad_rows(a2p), w2r, b2[None, :])
    sc2, sh2 = _bn_fold(s2, q2, g2, be2, float(N * NUM_ANGLE * NUM_RHO))

    w3r = W3.transpose(2, 3, 1, 0).reshape(9, DIM, DIM)
    o3, s3, q3 = _kconv3(_pad_rows(o2), w3r, b3[None, :], sc2, sh2)
    sc3, sh3 = _bn_fold(s3, q3, g3, be3, float(N * NUM_ANGLE * NUM_RHO))

    h3 = _kfinal(o3, sc3, sh3)
    h3f = h3.reshape(N, PG, PG, DIM)[:, 1:91, 1:91, :]
    return h3f.transpose(0, 3, 1, 2)
